# initial kernel scaffold (unmeasured)
import jax
import jax.numpy as jnp
from jax import lax
from jax.experimental import pallas as pl
from jax.experimental.pallas import tpu as pltpu


def kernel(
    x,
):
    def body(*refs):
        pass

    out_shape = jax.ShapeDtypeStruct(..., jnp.float32)
    return pl.pallas_call(body, out_shape=out_shape)(...)


# baseline (device time: 12463 ns/iter reference)
import jax
import jax.numpy as jnp
from jax import lax
from jax.experimental import pallas as pl
from jax.experimental.pallas import tpu as pltpu

N_DEV = 4


def kernel(x):
    _, m, n_total = x.shape
    n_per = n_total // N_DEV

    def body(x_ref, out_ref, recv_buf, send_sems, recv_sems):
        my_pos = lax.axis_index("i")

        barrier_sem = pltpu.get_barrier_semaphore()
        for k in range(1, N_DEV):
            peer = lax.rem(my_pos + k, N_DEV)
            pl.semaphore_signal(
                barrier_sem, inc=1,
                device_id=(peer,), device_id_type=pl.DeviceIdType.MESH,
            )
        pl.semaphore_wait(barrier_sem, N_DEV - 1)

        rdmas = []
        for k in range(1, N_DEV):
            dst = lax.rem(my_pos + k, N_DEV)
            slot = N_DEV - 1 - k
            rdma = pltpu.make_async_remote_copy(
                src_ref=x_ref.at[0, :, pl.ds(dst * n_per, n_per)],
                dst_ref=recv_buf.at[slot],
                send_sem=send_sems.at[slot],
                recv_sem=recv_sems.at[slot],
                device_id=(dst,),
                device_id_type=pl.DeviceIdType.MESH,
            )
            rdma.start()
            rdmas.append(rdma)

        acc = x_ref[0, :, pl.ds(my_pos * n_per, n_per)]

        for k in range(1, N_DEV):
            slot = N_DEV - 1 - k
            rdmas[k - 1].wait_recv()
            acc = acc + recv_buf[slot]
        out_ref[:, :] = acc

        for rdma in rdmas:
            rdma.wait_send()

    return pl.pallas_call(
        body,
        out_shape=jax.ShapeDtypeStruct((m, n_per), x.dtype),
        in_specs=[pl.BlockSpec(memory_space=pltpu.VMEM)],
        out_specs=pl.BlockSpec(memory_space=pltpu.VMEM),
        scratch_shapes=[
            pltpu.VMEM((N_DEV - 1, m, n_per), x.dtype),
            pltpu.SemaphoreType.DMA((N_DEV - 1,)),
            pltpu.SemaphoreType.DMA((N_DEV - 1,)),
        ],
        compiler_params=pltpu.CompilerParams(collective_id=0),
    )(x)


# device time: 11085 ns/iter; 1.1243x vs baseline; 1.1243x over previous
import jax
import jax.numpy as jnp
from jax import lax
from jax.experimental import pallas as pl
from jax.experimental.pallas import tpu as pltpu

N_DEV = 4

RAW_A = 0
RAW_B = 1
DIR_A = 2
DIR_B = 3
PART_A = 4
PART_B = 5


def kernel(x):
    _, m, n_total = x.shape
    n_per = n_total // N_DEV
    h = m // 2

    def body(x_ref, out_ref, recv_buf, stage_buf, send_sems, recv_sems):
        my = lax.axis_index("i")
        left = lax.rem(my + N_DEV - 1, N_DEV)
        right = lax.rem(my + 1, N_DEV)
        c_left = left * n_per
        c_right = right * n_per
        c_opp = lax.rem(my + 2, N_DEV) * n_per
        c_mine = my * n_per

        T = pl.ds(0, h)
        Bo = pl.ds(h, h)

        def copy(src, slot, dst_dev):
            return pltpu.make_async_remote_copy(
                src_ref=src,
                dst_ref=recv_buf.at[slot],
                send_sem=send_sems.at[slot],
                recv_sem=recv_sems.at[slot],
                device_id=(dst_dev,),
                device_id_type=pl.DeviceIdType.MESH,
            )

        barrier_sem = pltpu.get_barrier_semaphore()
        for nbr in [left, right]:
            pl.semaphore_signal(
                barrier_sem, inc=1,
                device_id=(nbr,), device_id_type=pl.DeviceIdType.MESH,
            )
        pl.semaphore_wait(barrier_sem, 2)

        raw_a = copy(x_ref.at[0, T, pl.ds(c_opp, n_per)], RAW_A, left)
        raw_b = copy(x_ref.at[0, Bo, pl.ds(c_opp, n_per)], RAW_B, right)
        dir_a = copy(x_ref.at[0, T, pl.ds(c_right, n_per)], DIR_A, right)
        dir_b = copy(x_ref.at[0, Bo, pl.ds(c_left, n_per)], DIR_B, left)
        raw_a.start()
        raw_b.start()
        dir_a.start()
        dir_b.start()

        raw_a.wait_recv()
        stage_buf[0] = x_ref[0, T, pl.ds(c_left, n_per)] + recv_buf[RAW_A]
        part_a = copy(stage_buf.at[0], PART_A, left)
        part_a.start()

        raw_b.wait_recv()
        stage_buf[1] = x_ref[0, Bo, pl.ds(c_right, n_per)] + recv_buf[RAW_B]
        part_b = copy(stage_buf.at[1], PART_B, right)
        part_b.start()

        dir_a.wait_recv()
        acc_t = x_ref[0, T, pl.ds(c_mine, n_per)] + recv_buf[DIR_A]
        dir_b.wait_recv()
        acc_b = x_ref[0, Bo, pl.ds(c_mine, n_per)] + recv_buf[DIR_B]
        part_a.wait_recv()
        out_ref[T, :] = acc_t + recv_buf[PART_A]
        part_b.wait_recv()
        out_ref[Bo, :] = acc_b + recv_buf[PART_B]

        for rdma in [raw_a, raw_b, dir_a, dir_b, part_a, part_b]:
            rdma.wait_send()

    return pl.pallas_call(
        body,
        out_shape=jax.ShapeDtypeStruct((m, n_per), x.dtype),
        in_specs=[pl.BlockSpec(memory_space=pltpu.VMEM)],
        out_specs=pl.BlockSpec(memory_space=pltpu.VMEM),
        scratch_shapes=[
            pltpu.VMEM((6, h, n_per), x.dtype),
            pltpu.VMEM((2, h, n_per), x.dtype),
            pltpu.SemaphoreType.DMA((6,)),
            pltpu.SemaphoreType.DMA((6,)),
        ],
        compiler_params=pltpu.CompilerParams(collective_id=0),
    )(x)
